# Initial kernel scaffold; baseline (speedup 1.0000x reference)
import jax
import jax.numpy as jnp
from jax.experimental import pallas as pl


def kernel(x, l, params):
    raise NotImplementedError("write your pallas kernel here")


# speedup vs baseline: 45.2409x; 45.2409x over previous
import functools

import jax
import jax.numpy as jnp
from jax import lax
from jax.experimental import pallas as pl
from jax.experimental.pallas import tpu as pltpu
from jax.experimental.pallas import tpu_sc as plsc

F32 = jnp.float32
I32 = jnp.int32
EPS = 1e-5
NW = 32
CH = 128


def _topk_body(k, exact, q_ref, p_ref, idx_ref, val_ref):
    q = q_ref[0]
    p = p_ref[0]
    C = q.shape[1]
    if exact and C <= 8:
        pd = jnp.zeros((q.shape[0], p.shape[0]), F32)
        for c in range(C):
            dc = q[:, c][:, None] - p[:, c][None, :]
            pd = pd - dc * dc
    else:
        qq = jnp.sum(q * q, axis=1, keepdims=True)
        pp = jnp.sum(p * p, axis=1)[None, :]
        g = lax.dot_general(q, p, (((1,), (1,)), ((), ())),
                            preferred_element_type=F32)
        pd = 2.0 * g - qq - pp
    tq, np_ = pd.shape
    iota = lax.broadcasted_iota(I32, (tq, np_), 1)
    neg = jnp.float32(-jnp.inf)
    for r in range(k):
        m = jnp.max(pd, axis=1, keepdims=True)
        sel = pd == m
        ji = jnp.min(jnp.where(sel, iota, np_), axis=1)
        idx_ref[0, r, :] = ji
        val_ref[0, r, :] = m[:, 0]
        pd = jnp.where(iota == ji[:, None], neg, pd)


def _topk(q, p, k, exact=False):
    B, Nq, C = q.shape
    Np = p.shape[1]
    TQ = min(256, Nq)
    grid = (B, Nq // TQ)
    return pl.pallas_call(
        functools.partial(_topk_body, k, exact),
        grid=grid,
        in_specs=[
            pl.BlockSpec((1, TQ, C), lambda b, t: (b, t, 0)),
            pl.BlockSpec((1, Np, C), lambda b, t: (b, 0, 0)),
        ],
        out_specs=[
            pl.BlockSpec((1, k, TQ), lambda b, t: (b, 0, t)),
            pl.BlockSpec((1, k, TQ), lambda b, t: (b, 0, t)),
        ],
        out_shape=[
            jax.ShapeDtypeStruct((B, k, Nq), I32),
            jax.ShapeDtypeStruct((B, k, Nq), F32),
        ],
    )(q, p)


def _apply_act(z, act):
    if act == 'relu':
        return jnp.maximum(z, 0.0)
    if act == 'lrelu':
        return jnp.where(z > 0, z, 0.2 * z)
    if act == 'sig_lrelu':
        z = jnp.where(z > 0, z, 0.2 * z)
        return 1.0 / (1.0 + jnp.exp(-z))
    return z


def _dense_body(act, has_norm, has_bias, has_bb, want_y, want_stats, want_max,
                *refs):
    refs = list(refs)
    x_ref = refs.pop(0)
    w_ref = refs.pop(0)
    m_ref = refs.pop(0) if has_norm else None
    s_ref = refs.pop(0) if has_norm else None
    b_ref = refs.pop(0) if has_bias else None
    bb_ref = refs.pop(0) if has_bb else None
    y_ref = refs.pop(0) if want_y else None
    ss_ref = refs.pop(0) if want_stats else None
    sq_ref = refs.pop(0) if want_stats else None
    mx_ref = refs.pop(0) if want_max else None

    t = pl.program_id(1)
    z = x_ref[0]
    if has_norm:
        z = (z - m_ref[0, 0][None, :]) * s_ref[0, 0][None, :]
    z = _apply_act(z, act)
    y = lax.dot_general(z, w_ref[...], (((1,), (1,)), ((), ())),
                        preferred_element_type=F32)
    if has_bias:
        y = y + b_ref[0, 0][None, :]
    if has_bb:
        y = y + bb_ref[0, 0][None, :]
    if want_y:
        y_ref[0] = y
    if want_stats:
        b = pl.program_id(0)
        @pl.when(jnp.logical_and(b == 0, t == 0))
        def _():
            ss_ref[0, 0] = jnp.zeros_like(ss_ref[0, 0])
            sq_ref[0, 0] = jnp.zeros_like(sq_ref[0, 0])
        ss_ref[0, 0] += jnp.sum(y, axis=0)
        sq_ref[0, 0] += jnp.sum(y * y, axis=0)
    if want_max:
        ym = jnp.max(y, axis=0)
        @pl.when(t == 0)
        def _():
            mx_ref[0, 0] = ym
        @pl.when(t != 0)
        def _():
            mx_ref[0, 0] = jnp.maximum(mx_ref[0, 0], ym)


def _dense(x, W, m=None, sinv=None, act='none', bias=None, bb=None,
           want_y=True, want_stats=False, want_max=False):
    B, N, Cin = x.shape
    Cout = W.shape[0]
    TN = min(256, N)
    has_norm = m is not None
    has_bias = bias is not None
    has_bb = bb is not None
    inputs = [x, W]
    in_specs = [pl.BlockSpec((1, TN, Cin), lambda b, t: (b, t, 0)),
                pl.BlockSpec((Cout, Cin), lambda b, t: (0, 0))]
    if has_norm:
        inputs += [m.reshape(1, 1, Cin), sinv.reshape(1, 1, Cin)]
        in_specs += [pl.BlockSpec((1, 1, Cin), lambda b, t: (0, 0, 0))] * 2
    if has_bias:
        inputs.append(bias.reshape(1, 1, Cout))
        in_specs.append(pl.BlockSpec((1, 1, Cout), lambda b, t: (0, 0, 0)))
    if has_bb:
        inputs.append(bb.reshape(B, 1, Cout))
        in_specs.append(pl.BlockSpec((1, 1, Cout), lambda b, t: (b, 0, 0)))
    out_specs, out_shape = [], []
    if want_y:
        out_specs.append(pl.BlockSpec((1, TN, Cout), lambda b, t: (b, t, 0)))
        out_shape.append(jax.ShapeDtypeStruct((B, N, Cout), F32))
    if want_stats:
        out_specs += [pl.BlockSpec((1, 1, Cout), lambda b, t: (0, 0, 0))] * 2
        out_shape += [jax.ShapeDtypeStruct((1, 1, Cout), F32)] * 2
    if want_max:
        out_specs.append(pl.BlockSpec((1, 1, Cout), lambda b, t: (b, 0, 0)))
        out_shape.append(jax.ShapeDtypeStruct((B, 1, Cout), F32))
    outs = pl.pallas_call(
        functools.partial(_dense_body, act, has_norm, has_bias, has_bb,
                          want_y, want_stats, want_max),
        grid=(B, N // TN),
        in_specs=in_specs, out_specs=out_specs, out_shape=out_shape,
    )(*inputs)
    return list(outs)


def _msmall_body(has_bias, want_stats, *refs):
    refs = list(refs)
    x_ref, w_ref = refs.pop(0), refs.pop(0)
    b_ref = refs.pop(0) if has_bias else None
    y_ref = refs.pop(0)
    y = lax.dot_general(x_ref[...], w_ref[...], (((1,), (1,)), ((), ())),
                        preferred_element_type=F32)
    if has_bias:
        y = y + b_ref[0][None, :]
    y_ref[...] = y
    if want_stats:
        ss_ref, sq_ref = refs
        ss_ref[0] = jnp.sum(y, axis=0)
        sq_ref[0] = jnp.sum(y * y, axis=0)


def _matmul_small(X, W, bias=None, want_stats=False):
    M, K = X.shape
    Cout = W.shape[0]
    inputs = [X, W]
    in_specs = [pl.BlockSpec((M, K), lambda: (0, 0)),
                pl.BlockSpec((Cout, K), lambda: (0, 0))]
    if bias is not None:
        inputs.append(bias.reshape(1, Cout))
        in_specs.append(pl.BlockSpec((1, Cout), lambda: (0, 0)))
    out_specs = [pl.BlockSpec((M, Cout), lambda: (0, 0))]
    out_shape = [jax.ShapeDtypeStruct((M, Cout), F32)]
    if want_stats:
        out_specs += [pl.BlockSpec((1, Cout), lambda: (0, 0))] * 2
        out_shape += [jax.ShapeDtypeStruct((1, Cout), F32)] * 2
    return pl.pallas_call(
        functools.partial(_msmall_body, bias is not None, want_stats),
        in_specs=in_specs, out_specs=out_specs, out_shape=out_shape,
    )(*inputs)


def _bnact_body(act, has_add, *refs):
    refs = list(refs)
    x_ref = refs.pop(0)
    a_ref = refs.pop(0) if has_add else None
    m_ref, s_ref, z_ref = refs
    z = x_ref[0]
    if has_add:
        z = z + a_ref[0]
    z = (z - m_ref[0, 0][None, :]) * s_ref[0, 0][None, :]
    z_ref[0] = _apply_act(z, act)


def _bn_act(x, add, m, sinv, act):
    B, N, C = x.shape
    TN = min(256, N)
    inputs = [x]
    in_specs = [pl.BlockSpec((1, TN, C), lambda b, t: (b, t, 0))]
    if add is not None:
        inputs.append(add)
        in_specs.append(pl.BlockSpec((1, TN, C), lambda b, t: (b, t, 0)))
    inputs += [m.reshape(1, 1, C), sinv.reshape(1, 1, C)]
    in_specs += [pl.BlockSpec((1, 1, C), lambda b, t: (0, 0, 0))] * 2
    return pl.pallas_call(
        functools.partial(_bnact_body, act, add is not None),
        grid=(B, N // TN),
        in_specs=in_specs,
        out_specs=pl.BlockSpec((1, TN, C), lambda b, t: (b, t, 0)),
        out_shape=jax.ShapeDtypeStruct((B, N, C), F32),
    )(*inputs)


def _csums_body(s_ref, q_ref, b_ref, o1_ref, o2_ref, o3_ref):
    i = pl.program_id(0)
    @pl.when(i == 0)
    def _():
        o1_ref[0] = jnp.zeros_like(o1_ref[0])
        o2_ref[0] = jnp.zeros_like(o2_ref[0])
        o3_ref[0] = jnp.zeros_like(o3_ref[0])
    s = s_ref[...]
    o1_ref[0] += jnp.sum(s, axis=0)
    o2_ref[0] += jnp.sum(q_ref[...], axis=0)
    o3_ref[0] += jnp.sum(b_ref[...] * s, axis=0)


def _chan_sums(S, Q, Bp):
    R, C = S.shape
    TR = min(512, R)
    return pl.pallas_call(
        _csums_body,
        grid=(R // TR,),
        in_specs=[pl.BlockSpec((TR, C), lambda i: (i, 0))] * 3,
        out_specs=[pl.BlockSpec((1, C), lambda i: (0, 0))] * 3,
        out_shape=[jax.ShapeDtypeStruct((1, C), F32)] * 3,
    )(S, Q, Bp)


def _gf_body(K, TP, C, mode, *refs):
    refs = list(refs)
    e_ref = refs.pop(0)
    x_ref = refs.pop(0)
    w1_ref = refs.pop(0)
    if mode == 'conv2':
        m_ref = refs.pop(0)
        s_ref = refs.pop(0)
        w2_ref = refs.pop(0)
    if mode != 'stats':
        mx_ref = refs.pop(0)
    ss_ref = refs.pop(0)
    sq_ref = refs.pop(0)

    i = pl.program_id(0)
    xj = e_ref[...][:, :C].reshape(TP, K, C)
    xn = x_ref[...][:, :C]
    gf = jnp.concatenate(
        [xj - xn[:, None, :], jnp.broadcast_to(xn[:, None, :], (TP, K, C))],
        axis=2).reshape(TP * K, 2 * C)
    e1 = lax.dot_general(gf, w1_ref[...], (((1,), (1,)), ((), ())),
                         preferred_element_type=F32)
    if mode == 'conv2':
        t1 = (e1 - m_ref[0][None, :]) * s_ref[0][None, :]
        t1 = jnp.where(t1 > 0, t1, 0.2 * t1)
        e1 = lax.dot_general(t1, w2_ref[...], (((1,), (1,)), ((), ())),
                             preferred_element_type=F32)
    if mode != 'stats':
        mx_ref[...] = jnp.max(e1.reshape(TP, K, 64), axis=1)
    @pl.when(i == 0)
    def _():
        ss_ref[0] = jnp.zeros_like(ss_ref[0])
        sq_ref[0] = jnp.zeros_like(sq_ref[0])
    ss_ref[0] += jnp.sum(e1, axis=0)
    sq_ref[0] += jnp.sum(e1 * e1, axis=0)


def _gf_conv(edges, xrows, W1, K, mode, m1=None, s1inv=None, W2=None):
    R, C = xrows.shape
    TP = 64 if R * K >= 8192 else max(8, 2048 // K)
    while R % TP:
        TP //= 2
    inputs = [edges, xrows, W1]
    in_specs = [pl.BlockSpec((TP * K, 128), lambda i: (i, 0)),
                pl.BlockSpec((TP, C), lambda i: (i, 0)),
                pl.BlockSpec(W1.shape, lambda i: (0, 0))]
    if mode == 'conv2':
        inputs += [m1.reshape(1, 64), s1inv.reshape(1, 64), W2]
        in_specs += [pl.BlockSpec((1, 64), lambda i: (0, 0))] * 2
        in_specs += [pl.BlockSpec((64, 64), lambda i: (0, 0))]
    out_specs, out_shape = [], []
    if mode != 'stats':
        out_specs.append(pl.BlockSpec((TP, 64), lambda i: (i, 0)))
        out_shape.append(jax.ShapeDtypeStruct((R, 64), F32))
    out_specs += [pl.BlockSpec((1, 64), lambda i: (0, 0))] * 2
    out_shape += [jax.ShapeDtypeStruct((1, 64), F32)] * 2
    return pl.pallas_call(
        functools.partial(_gf_body, K, TP, C, mode),
        grid=(R // TP,),
        in_specs=in_specs, out_specs=out_specs, out_shape=out_shape,
    )(*inputs)


def _coldot_body(a_ref, b_ref, c_ref, o_ref):
    s = jnp.sum(a_ref[0] * b_ref[0], axis=1) + c_ref[0, 0]
    o_ref[0, 0] = 1.0 / (1.0 + jnp.exp(-s))


def _coldot(a, b, bias_scalar):
    B, N, C = a.shape
    TN = min(256, N)
    return pl.pallas_call(
        _coldot_body,
        grid=(B, N // TN),
        in_specs=[
            pl.BlockSpec((1, TN, C), lambda bb, t: (bb, t, 0)),
            pl.BlockSpec((1, TN, C), lambda bb, t: (bb, t, 0)),
            pl.BlockSpec((1, 1), lambda bb, t: (0, 0)),
        ],
        out_specs=pl.BlockSpec((1, 1, TN), lambda bb, t: (bb, 0, t)),
        out_shape=jax.ShapeDtypeStruct((B, 1, N), F32),
    )(a, b, bias_scalar.reshape(1, 1))


def _rank_body(kp, TS, s_ref, seq_ref, xyz_ref, so_ref, xo_ref, vo_ref):
    t = pl.program_id(1)
    nT = pl.num_programs(1)
    sfull = s_ref[0, 0, :]
    N = sfull.shape[0]
    st = s_ref[0, 0, pl.ds(t * TS, TS)]
    @pl.when(t == 0)
    def _():
        so_ref[0] = jnp.zeros_like(so_ref[0])
        xo_ref[0] = jnp.zeros_like(xo_ref[0])
        vo_ref[0, 0] = jnp.zeros_like(vo_ref[0, 0])
    si = st[:, None]
    sj = sfull[None, :]
    jj = lax.broadcasted_iota(I32, (TS, N), 1)
    gi = t * TS + lax.broadcasted_iota(I32, (TS, N), 0)
    cnt = jnp.sum((jnp.logical_or(sj > si,
                                  jnp.logical_and(sj == si, jj < gi))
                   ).astype(I32), axis=1)
    rr = lax.broadcasted_iota(I32, (kp, TS), 0)
    ohb = rr == cnt[None, :]
    oh = ohb.astype(F32)
    vo_ref[0, 0] += jnp.sum(jnp.where(ohb, st[None, :], 0.0), axis=1)
    so_ref[0] += lax.dot_general(oh, seq_ref[0], (((1,), (0,)), ((), ())),
                                 preferred_element_type=F32,
                                 precision=lax.Precision.HIGHEST)
    xo_ref[0] += lax.dot_general(oh, xyz_ref[0], (((1,), (0,)), ((), ())),
                                 preferred_element_type=F32,
                                 precision=lax.Precision.HIGHEST)
    @pl.when(t == nT - 1)
    def _():
        v = vo_ref[0, 0][:, None]
        so_ref[0] *= v
        xo_ref[0] *= v


def _rank_select(scores, seq, xyz, kp):
    B, N, C = seq.shape
    Cx = xyz.shape[2]
    TS = min(256, N)
    return pl.pallas_call(
        functools.partial(_rank_body, kp, TS),
        grid=(B, N // TS),
        in_specs=[
            pl.BlockSpec((1, 1, N), lambda b, t: (b, 0, 0)),
            pl.BlockSpec((1, TS, C), lambda b, t: (b, t, 0)),
            pl.BlockSpec((1, TS, Cx), lambda b, t: (b, t, 0)),
        ],
        out_specs=[
            pl.BlockSpec((1, kp, C), lambda b, t: (b, 0, 0)),
            pl.BlockSpec((1, kp, Cx), lambda b, t: (b, 0, 0)),
            pl.BlockSpec((1, 1, kp), lambda b, t: (b, 0, 0)),
        ],
        out_shape=[
            jax.ShapeDtypeStruct((B, kp, C), F32),
            jax.ShapeDtypeStruct((B, kp, Cx), F32),
            jax.ShapeDtypeStruct((B, 1, kp), F32),
        ],
    )(scores, seq, xyz)


def _sc_mesh():
    return plsc.VectorSubcoreMesh(core_axis_name="c", subcore_axis_name="s")


def _reduce_chunk(K, cpts, buf, stage):
    def pbody(p, _):
        base = p * K
        inits = []
        for g in range(4):
            v = buf[base, pl.ds(g * 16, 16)]
            inits += [v, v, v * v]
        def kbody(kk, acc):
            out = []
            for g in range(4):
                v = buf[base + kk, pl.ds(g * 16, 16)]
                m, s, q = acc[3 * g], acc[3 * g + 1], acc[3 * g + 2]
                out += [jnp.maximum(m, v), s + v, q + v * v]
            return tuple(out)
        acc = lax.fori_loop(1, K, kbody, tuple(inits))
        for g in range(4):
            stage[p, pl.ds(g * 16, 16)] = acc[3 * g]
            stage[p, pl.ds(64 + g * 16, 16)] = acc[3 * g + 1]
            stage[p, pl.ds(128 + g * 16, 16)] = acc[3 * g + 2]
        return 0
    lax.fori_loop(0, cpts, pbody, 0)


def _chunk_rows(ppw, K):
    if (ppw * K) % 256 == 0 and ppw % (256 // K) == 0 and K <= 32:
        return 256
    return 128


def _sc_gather_reduce(table, idx3, n_pts, K, dump):
    E = n_pts * K
    ppw = n_pts // NW
    ch = _chunk_rows(ppw, K)
    S = ch // 128
    cpts = ch // K
    nck = ppw // cpts
    assert n_pts % NW == 0 and ppw % cpts == 0 and ch % K == 0
    assert idx3.shape == (NW, nck * S, 128)

    out_type = [jax.ShapeDtypeStruct((n_pts, 256), F32)]
    if dump:
        out_type.append(jax.ShapeDtypeStruct((E, 128), F32))
    scratch = [
        pltpu.VMEM((nck * S, 128), I32),
        pltpu.VMEM((ch, 128), F32), pltpu.VMEM((ch, 128), F32),
        pltpu.VMEM((cpts, 256), F32), pltpu.VMEM((cpts, 256), F32),
        pltpu.SemaphoreType.DMA, pltpu.SemaphoreType.DMA,
        pltpu.SemaphoreType.DMA, pltpu.SemaphoreType.DMA,
    ]

    def body(table_h, idx_h, *rest):
        rest = list(rest)
        packed_h = rest.pop(0)
        edges_h = rest.pop(0) if dump else None
        idx_v, rows0, rows1, st0, st1, g0, g1, w0, w1 = rest
        wid = lax.axis_index("s") * 2 + lax.axis_index("c")
        pltpu.sync_copy(idx_h.at[wid], idx_v)
        p0 = wid * ppw
        r0 = p0 * K

        def start_gather(c, rows, sem):
            for s in range(S):
                pltpu.make_async_copy(
                    table_h.at[idx_v.at[c * S + s]],
                    rows.at[pl.ds(s * 128, 128)], sem).start()

        def wait_gather(c, rows, sem):
            for s in range(S):
                pltpu.make_async_copy(
                    table_h.at[idx_v.at[c * S + s]],
                    rows.at[pl.ds(s * 128, 128)], sem).wait()

        def handle(c, rows, stage, gsem, wsem):
            wait_gather(c, rows, gsem)
            @pl.when(c >= 2)
            def _():
                pltpu.make_async_copy(
                    stage, packed_h.at[pl.ds(p0, cpts)], wsem).wait()
            _reduce_chunk(K, cpts, rows, stage)
            if dump:
                pltpu.sync_copy(rows, edges_h.at[pl.ds(r0 + c * ch, ch)])
            @pl.when(c + 2 < nck)
            def _():
                start_gather(c + 2, rows, gsem)
            pltpu.make_async_copy(
                stage, packed_h.at[pl.ds(p0 + c * cpts, cpts)], wsem).start()

        if nck == 1:
            start_gather(0, rows0, g0)
            wait_gather(0, rows0, g0)
            _reduce_chunk(K, cpts, rows0, st0)
            if dump:
                pltpu.sync_copy(rows0, edges_h.at[pl.ds(r0, ch)])
            pltpu.sync_copy(st0, packed_h.at[pl.ds(p0, cpts)])
        else:
            start_gather(0, rows0, g0)
            start_gather(1, rows1, g1)
            def pair(i, _):
                c = 2 * i
                handle(c, rows0, st0, g0, w0)
                handle(c + 1, rows1, st1, g1, w1)
                return 0
            lax.fori_loop(0, nck // 2, pair, 0)
            pltpu.make_async_copy(st0, packed_h.at[pl.ds(p0, cpts)], w0).wait()
            pltpu.make_async_copy(st1, packed_h.at[pl.ds(p0, cpts)], w1).wait()

    return pl.kernel(body, out_type=out_type, mesh=_sc_mesh(),
                     scratch_types=scratch)(table, idx3)


def _sc_gather_dump(table, idx3, E):
    rpw = E // NW
    nck = rpw // CH
    assert E % NW == 0 and rpw % CH == 0
    assert idx3.shape == (NW, nck, 128)
    scratch = [
        pltpu.VMEM((nck, 128), I32),
        pltpu.VMEM((CH, 128), F32), pltpu.VMEM((CH, 128), F32),
        pltpu.SemaphoreType.DMA, pltpu.SemaphoreType.DMA,
    ]

    def body(table_h, idx_h, edges_h, idx_v, rows0, rows1, g0, g1):
        wid = lax.axis_index("s") * 2 + lax.axis_index("c")
        pltpu.sync_copy(idx_h.at[wid], idx_v)
        r0 = wid * rpw

        def start_gather(c, rows, sem):
            pltpu.make_async_copy(table_h.at[idx_v.at[c]], rows, sem).start()

        def handle(c, rows, gsem):
            pltpu.make_async_copy(table_h.at[idx_v.at[c]], rows, gsem).wait()
            pltpu.sync_copy(rows, edges_h.at[pl.ds(r0 + c * CH, CH)])
            @pl.when(c + 2 < nck)
            def _():
                start_gather(c + 2, rows, gsem)

        if nck == 1:
            start_gather(0, rows0, g0)
            handle(0, rows0, g0)
        else:
            start_gather(0, rows0, g0)
            start_gather(1, rows1, g1)
            def pair(i, _):
                handle(2 * i, rows0, g0)
                handle(2 * i + 1, rows1, g1)
                return 0
            lax.fori_loop(0, nck // 2, pair, 0)

    return pl.kernel(body,
                     out_type=[jax.ShapeDtypeStruct((E, 128), F32)],
                     mesh=_sc_mesh(), scratch_types=scratch)(table, idx3)


def _flat_idx(idx_knq, Np):
    B, k, Nq = idx_knq.shape
    g = jnp.transpose(idx_knq, (0, 2, 1))
    g = g + (jnp.arange(B, dtype=I32) * Np)[:, None, None]
    return g.reshape(NW, -1, 128)


def _pad_table(A2d):
    return jnp.pad(A2d, ((0, 0), (0, 64)))


def _finalize(ssum, ssq, count):
    m = ssum.reshape(-1) / count
    v = ssq.reshape(-1) / count - m * m
    return m, 1.0 / jnp.sqrt(v + EPS)


def _edge_stats(cS, cQ, cBS, sB, qB, R, K):
    tot = R * K
    m = (cS.reshape(-1) + K * sB.reshape(-1)) / tot
    e2 = (cQ.reshape(-1) + 2.0 * cBS.reshape(-1) + K * qB.reshape(-1)) / tot
    v = e2 - m * m
    return m, 1.0 / jnp.sqrt(v + EPS)


def _unpack_msq(packed, B, N):
    p = packed.reshape(B, N, 256)
    return p[:, :, :64], p[:, :, 64:128], p[:, :, 128:192]


def _pad128(t):
    R, C = t.shape
    return t if C == 128 else jnp.pad(t, ((0, 0), (0, 128 - C)))


def _single_edgeconv(x, W, k, idx=None):
    B, N, C = x.shape
    if idx is None:
        idx, _ = _topk(x, x, k)
    xr = x.reshape(B * N, C)
    edges = _sc_gather_dump(_pad128(xr), _flat_idx(idx, N), B * N * k)[0]
    mx, ss, sq = _gf_conv(edges, xr, W, k, 'single')
    m, sinv = _finalize(ss, sq, B * N * k)
    return mx.reshape(B, N, 64), m, sinv


def _double_edgeconv(x, W1, W2, k):
    B, N, C = x.shape
    idx, _ = _topk(x, x, k)
    xr = x.reshape(B * N, C)
    edges = _sc_gather_dump(_pad128(xr), _flat_idx(idx, N), B * N * k)[0]
    ss1, sq1 = _gf_conv(edges, xr, W1, k, 'stats')
    m1, s1inv = _finalize(ss1, sq1, B * N * k)
    mx, ss2, sq2 = _gf_conv(edges, xr, W1, k, 'conv2', m1, s1inv, W2)
    m2, s2inv = _finalize(ss2, sq2, B * N * k)
    return _bn_act(mx.reshape(B, N, 64), None, m2, s2inv, 'lrelu')


def _index_select(xyz, seq, p, kp, k):
    B, N, _ = seq.shape
    yfc, sfc, qfc = _dense(seq, p['fc_W'], bias=p['fc_b'], want_stats=True)
    mfc, sifc = _finalize(sfc, qfc, B * N)
    h1 = _bn_act(yfc, None, mfc, sifc, 'relu')
    mxE, mE, siE = _single_edgeconv(h1, p['ec_W'], k)
    X = _bn_act(mxE, None, mE, siE, 'sig_lrelu')
    g = _dense(h1, p['disc_W'].T)[0]
    scores = _coldot(g, X, p['disc_b'].reshape(1,))
    seq_new, xyz_new, _ = _rank_select(scores, seq, xyz, kp)
    return seq_new, xyz_new


def _aggregate(xyz, node, feats, k):
    B, N, _ = feats.shape
    Nn = node.shape[1]
    idx, _ = _topk(node, xyz, k, exact=True)
    packed = _sc_gather_reduce(_pad_table(feats.reshape(B * N, 64)),
                               _flat_idx(idx, N), B * Nn, k, dump=False)[0]
    return packed.reshape(B, Nn, 256)[:, :, :64]


def _unpool(xyz_src, xyz_unknown, feats):
    B, Ns, _ = feats.shape
    Nu = xyz_unknown.shape[1]
    ni, negv = _topk(xyz_unknown, xyz_src, 3, exact=True)
    w = jax.nn.softmax(-jnp.transpose(negv, (0, 2, 1)), axis=-1)
    rows = _sc_gather_dump(_pad_table(feats.reshape(B * Ns, 64)),
                           _flat_idx(ni, Ns), B * Nu * 3)[0]
    fu = rows[:, :64].reshape(B, Nu, 3, 64)
    return jnp.sum(w[:, :, :, None] * fu, axis=2)


def kernel(x, l, params):
    B, _, N = x.shape
    K = 32
    xt = jnp.transpose(x, (0, 2, 1))

    x1 = _double_edgeconv(xt, params['W1'], params['W2'], K)

    seq1, xyz1 = _index_select(xt, x1, params['p1'], N // 4, K // 2)
    agg1 = _aggregate(xt, xyz1, x1, K)
    f2 = jnp.concatenate([seq1, agg1], axis=2)

    x2 = _double_edgeconv(f2, params['W3'], params['W4'], K)

    seq2, xyz2 = _index_select(xyz1, x2, params['p2'], N // 16, K // 4)
    agg2 = _aggregate(xyz1, xyz2, x2, K)
    f3 = jnp.concatenate([seq2, agg2], axis=2)

    mx5, m5, si5 = _single_edgeconv(f3, params['W5'], K)
    x3 = _bn_act(mx5, None, m5, si5, 'lrelu')

    seq3, xyz3 = _index_select(xyz2, x3, params['p3'], N // 64, K // 8)
    agg3 = _aggregate(xyz2, xyz3, x3, K)
    f4 = jnp.concatenate([seq3, agg3], axis=2)

    N4 = N // 64
    idxL4 = jnp.broadcast_to(jnp.arange(N4, dtype=I32)[None, :, None],
                             (B, N4, N4))
    mx6, m6, si6 = _single_edgeconv(f4, params['W6'], N4, idx=idxL4)
    x4 = _bn_act(mx6, None, m6, si6, 'lrelu')

    u2 = _unpool(xyz1, xt, x2)
    u3 = _unpool(xyz2, xt, x3)
    u4 = _unpool(xyz3, xt, x4)
    feats = jnp.concatenate([x1, u2, u3, u4], axis=2)

    sE, qE, ymE = _dense(feats, params['W6m'], want_y=False,
                         want_stats=True, want_max=True)
    mE, siE = _finalize(sE, qE, B * N)
    gz = (ymE[:, 0, :] - mE[None, :]) * siE[None, :]
    g = jnp.where(gz > 0, gz, 0.2 * gz)

    yl, sl, ql = _matmul_small(l[:, :, 0], params['W7'], want_stats=True)
    ml, sil = _finalize(sl, ql, B)
    lfz = (yl - ml[None, :]) * sil[None, :]
    lf = jnp.where(lfz > 0, lfz, 0.2 * lfz)

    gcat = jnp.concatenate([g, lf], axis=1)
    W8 = params['W8']
    P8a = _matmul_small(gcat, W8[:, :1088])[0]
    d, s8, q8 = _dense(feats, W8[:, 1088:], bb=P8a, want_stats=True)
    m8, si8 = _finalize(s8, q8, B * N)
    d, s9, q9 = _dense(d, params['W9'], m=m8, sinv=si8, act='lrelu',
                       want_stats=True)
    m9, si9 = _finalize(s9, q9, B * N)
    d, s10, q10 = _dense(d, params['W10'], m=m9, sinv=si9, act='lrelu',
                         want_stats=True)
    m10, si10 = _finalize(s10, q10, B * N)
    d, s11, q11 = _dense(d, params['W11'], m=m10, sinv=si10, act='lrelu',
                         want_stats=True)
    m11, si11 = _finalize(s11, q11, B * N)
    d, s12, q12 = _dense(d, params['W12'], m=m11, sinv=si11, act='lrelu',
                         want_stats=True)
    m12, si12 = _finalize(s12, q12, B * N)
    out = _dense(d, params['Wf'], m=m12, sinv=si12, act='lrelu',
                 bias=params['bf'])[0]
    return jnp.transpose(out, (0, 2, 1))
